# baseline (speedup 1.0000x reference)
import functools

import jax
import jax.numpy as jnp
from jax import lax
from jax.experimental import pallas as pl
from jax.experimental.pallas import tpu as pltpu
from jax.experimental.pallas import tpu_sc as plsc

PACKET = 64
NPK = 64
ROWLEN = NPK * PACKET


def _make_sc_sort(n_rows):
    info = plsc.get_sparse_core_info()
    NC, NS = info.num_cores, info.num_subcores
    NW = NC * NS
    rpw = n_rows // NW
    G = 4
    GL = G * ROWLEN
    NG = rpw // G
    mesh = plsc.VectorSubcoreMesh(core_axis_name="c", subcore_axis_name="s")

    @functools.partial(
        pl.kernel,
        mesh=mesh,
        out_type=jax.ShapeDtypeStruct((n_rows * ROWLEN,), jnp.float32),
        compiler_params=pltpu.CompilerParams(needs_layout_passes=False),
        scratch_types=[
            pltpu.VMEM((GL,), jnp.float32),
            pltpu.VMEM((GL,), jnp.float32),
            pltpu.VMEM((GL,), jnp.float32),
            pltpu.VMEM((GL,), jnp.float32),
            pltpu.VMEM((NPK,), jnp.float32),
            pltpu.VMEM((NPK,), jnp.int32),
            pltpu.SemaphoreType.DMA,
            pltpu.SemaphoreType.DMA,
            pltpu.SemaphoreType.DMA,
            pltpu.SemaphoreType.DMA,
        ],
    )
    def sortk(z_hbm, out_hbm, in0, in1, ou0, ou1, keys_v, rank_v,
              isem0, isem1, osem0, osem1):
        wid = lax.axis_index("s") * NC + lax.axis_index("c")
        base = wid * rpw * ROWLEN
        lanes = lax.iota(jnp.int32, 16)
        inb = (in0, in1)
        oub = (ou0, ou1)
        isems = (isem0, isem1)
        osems = (osem0, osem1)

        def in_cp(g, slot):
            return pltpu.make_async_copy(
                z_hbm.at[pl.ds(base + g * GL, GL)], inb[slot], isems[slot])

        def out_cp(g, slot):
            return pltpu.make_async_copy(
                oub[slot], out_hbm.at[pl.ds(base + g * GL, GL)], osems[slot])

        def process(slot):
            ib = inb[slot]
            ob = oub[slot]
            for rr in range(G):
                roff = rr * ROWLEN
                kv = []
                for v in range(4):
                    kvec = plsc.load_gather(
                        ib, [lanes * PACKET + (roff + v * 16 * PACKET)])
                    keys_v[pl.ds(v * 16, 16)] = kvec
                    kv.append(kvec)
                iv = [lanes + 16 * v for v in range(4)]

                def jbody(j, accs):
                    kjb = plsc.load_gather(keys_v, [jnp.full((16,), j)])
                    out = []
                    for v in range(4):
                        lt = kjb < kv[v]
                        tie = (kjb == kv[v]) & (j < iv[v])
                        out.append(accs[v]
                                   + jnp.where(lt | tie, 1, 0).astype(jnp.int32))
                    return tuple(out)

                accs = lax.fori_loop(
                    0, NPK, jbody,
                    tuple(jnp.zeros((16,), jnp.int32) for _ in range(4)))
                for v in range(4):
                    rank_v[pl.ds(v * 16, 16)] = accs[v]

                def pbody(i, _):
                    r = plsc.load_gather(rank_v, [jnp.full((16,), i)])[0]
                    src = roff + i * PACKET
                    dst = roff + r * PACKET
                    for t in range(4):
                        ob[pl.ds(dst + t * 16, 16)] = ib[pl.ds(src + t * 16, 16)]
                    return 0

                lax.fori_loop(0, NPK, pbody, 0)

        in_cp(0, 0).start()
        in_cp(1, 1).start()
        in_cp(0, 0).wait()
        process(0)
        out_cp(0, 0).start()
        in_cp(2, 0).start()
        in_cp(1, 1).wait()
        process(1)
        out_cp(1, 1).start()
        in_cp(3, 1).start()

        def body(p, _):
            for slot in range(2):
                g = 2 * p + slot
                in_cp(g, slot).wait()
                out_cp(g - 2, slot).wait()
                process(slot)
                out_cp(g, slot).start()
                in_cp(g + 2, slot).start()
            return 0

        lax.fori_loop(1, NG // 2 - 1, body, 0)

        for slot in range(2):
            g = NG - 2 + slot
            in_cp(g, slot).wait()
            out_cp(g - 2, slot).wait()
            process(slot)
            out_cp(g, slot).start()
        out_cp(NG - 2, 0).wait()
        out_cp(NG - 1, 1).wait()

    return sortk


def kernel(z):
    B, _, D = z.shape
    out = _make_sc_sort(B)(z.reshape(B * D))
    return out.reshape(B, 1, D)
